# baseline (speedup 1.0000x reference)
import functools

import jax
import jax.numpy as jnp
from jax import lax
from jax.experimental import pallas as pl
from jax.experimental.pallas import tpu as pltpu
from jax.experimental.pallas import tpu_sc as plsc

N_NODES = 10000
N_EDGES = 320000
D = 128

NC, NS = 2, 16
NW = NC * NS
EPW = N_EDGES // NW
CHUNK = 80
NCHUNK = EPW // CHUNK
NBUF = 3
L = 16
ZC = 40
NZC = N_NODES // ZC
ZITER = -(-NZC // NS)


def _sc_body(x_hbm, edges_hbm, z128_hbm,
             agg_out, deg_out,
             agg_sh, idx_v, rows_v, deg_v, z128_v, gsem, ssem, isem):
    cid = lax.axis_index("c")
    sid = lax.axis_index("s")
    wid = cid * NS + sid

    pltpu.sync_copy(z128_hbm, z128_v)

    zeros16 = jnp.zeros((L,), jnp.float32)

    def dzero_step(k, carry):
        deg_v[pl.ds(k * L, L)] = zeros16
        return carry

    lax.fori_loop(0, N_NODES // L, dzero_step, 0)

    def zero_step(k, carry):
        c = sid + k * NS

        @pl.when(c < NZC)
        def _():
            pltpu.sync_copy(z128_v, agg_sh.at[pl.ds(c * ZC, ZC)])

        return carry

    lax.fori_loop(0, ZITER, zero_step, 0)
    plsc.subcore_barrier()

    ones16 = jnp.ones((L,), jnp.float32)

    def start_idx(i, c):
        pltpu.async_copy(edges_hbm.at[wid, i], idx_v.at[c], isem.at[c])

    def wait_idx(i, c):
        pltpu.make_async_copy(
            edges_hbm.at[wid, i], idx_v.at[c], isem.at[c]).wait()

    def start_gather(b, c):
        pltpu.async_copy(x_hbm.at[idx_v.at[c, 0]], rows_v.at[b], gsem.at[b])

    def wait_gather(b, c):
        pltpu.make_async_copy(
            x_hbm.at[idx_v.at[c, 0]], rows_v.at[b], gsem.at[b]).wait()

    def start_scatter(b, c):
        pltpu.async_copy(rows_v.at[b], agg_sh.at[idx_v.at[c, 1]],
                         ssem.at[b], add=True)

    def wait_scatter(b, c):
        pltpu.make_async_copy(
            rows_v.at[b], agg_sh.at[idx_v.at[c, 1]], ssem.at[b]).wait()

    pltpu.sync_copy(edges_hbm.at[wid, 0], idx_v.at[0])
    start_idx(1, 1)
    start_idx(2, 2)
    start_gather(0, 0)

    UNROLL = 6

    def hex_step(j, carry):
        for p in range(UNROLL):
            i = UNROLL * j + p
            b, c = p % NBUF, p % 6

            @pl.when(i < NCHUNK)
            def _():
                for g in range(CHUNK // L):
                    dst16 = idx_v[c, 1, pl.ds(g * L, L)]
                    plsc.addupdate_scatter(deg_v, [dst16], ones16)
                wait_gather(b, c)
                start_scatter(b, c)

                @pl.when(i + 3 < NCHUNK)
                def _():
                    start_idx(i + 3, (p + 3) % 6)

                inext = i + 1

                @pl.when(inext < NCHUNK)
                def _():
                    b2, c2 = (p + 1) % NBUF, (p + 1) % 6

                    @pl.when(inext >= NBUF)
                    def _():
                        wait_scatter(b2, (p + 4) % 6)

                    wait_idx(inext, c2)
                    start_gather(b2, c2)

        return carry

    lax.fori_loop(0, -(-NCHUNK // UNROLL), hex_step, 0)
    wait_scatter((NCHUNK - 1) % NBUF, (NCHUNK - 1) % 6)
    wait_scatter((NCHUNK - 2) % NBUF, (NCHUNK - 2) % 6)
    wait_scatter((NCHUNK - 3) % NBUF, (NCHUNK - 3) % 6)
    plsc.subcore_barrier()

    pltpu.sync_copy(deg_v, deg_out.at[wid])

    def out_step(k, carry):
        c = sid + k * NS

        @pl.when(c < NZC)
        def _():
            r0 = c * ZC
            pltpu.sync_copy(agg_sh.at[pl.ds(r0, ZC)], z128_v)
            pltpu.sync_copy(z128_v, agg_out.at[cid, pl.ds(r0, ZC)])

        return carry

    lax.fori_loop(0, ZITER, out_step, 0)


_sc_call = pl.kernel(
    _sc_body,
    out_type=(jax.ShapeDtypeStruct((NC, N_NODES, D), jnp.float32),
              jax.ShapeDtypeStruct((NW, N_NODES), jnp.float32)),
    mesh=plsc.VectorSubcoreMesh(core_axis_name="c", subcore_axis_name="s",
                                num_cores=NC, num_subcores=NS),
    compiler_params=pltpu.CompilerParams(needs_layout_passes=False),
    scratch_types=[
        pltpu.VMEM_SHARED((N_NODES, D), jnp.float32),
        pltpu.VMEM((6, 2, CHUNK), jnp.int32),
        pltpu.VMEM((NBUF, CHUNK, D), jnp.float32),
        pltpu.VMEM((N_NODES,), jnp.float32),
        pltpu.VMEM((ZC, D), jnp.float32),
        pltpu.SemaphoreType.DMA((NBUF,)),
        pltpu.SemaphoreType.DMA((NBUF,)),
        pltpu.SemaphoreType.DMA((6,)),
    ],
)


BLK = 512


def _tc_body(x_ref, a_ref, d_ref, wst_ref, wnt_ref, b_ref, o_ref):
    x = x_ref[...]
    agg = a_ref[0] + a_ref[1]
    deg = jnp.sum(d_ref[...], axis=0)[:, None]
    deg = jnp.maximum(deg, 1.0)
    agg = agg / deg
    h = jnp.dot(x, wst_ref[...], preferred_element_type=jnp.float32)
    h = h + jnp.dot(agg, wnt_ref[...], preferred_element_type=jnp.float32)
    h = h + b_ref[...]
    o_ref[...] = jnp.maximum(h, 0.0)


_tc_call = pl.pallas_call(
    _tc_body,
    grid=(-(-N_NODES // BLK),),
    in_specs=[
        pl.BlockSpec((BLK, D), lambda i: (i, 0)),
        pl.BlockSpec((NC, BLK, D), lambda i: (0, i, 0)),
        pl.BlockSpec((NW, BLK), lambda i: (0, i)),
        pl.BlockSpec((D, D), lambda i: (0, 0)),
        pl.BlockSpec((D, D), lambda i: (0, 0)),
        pl.BlockSpec((1, D), lambda i: (0, 0)),
    ],
    out_specs=pl.BlockSpec((BLK, D), lambda i: (i, 0)),
    out_shape=jax.ShapeDtypeStruct((N_NODES, D), jnp.float32),
)


def kernel(x, edge_index, W_self, b_self, W_neigh, b_neigh):
    ei = edge_index.astype(jnp.int32)
    edges = ei.reshape(2, NW, NCHUNK, CHUNK).transpose(1, 2, 0, 3)
    z128 = jnp.zeros((ZC, D), jnp.float32)
    agg_parts, deg_parts = _sc_call(x, edges, z128)
    bias = (b_self + b_neigh)[None, :]
    return _tc_call(x, agg_parts, deg_parts, W_self.T, W_neigh.T, bias)


# speedup vs baseline: 1.0113x; 1.0113x over previous
import functools

import jax
import jax.numpy as jnp
from jax import lax
from jax.experimental import pallas as pl
from jax.experimental.pallas import tpu as pltpu
from jax.experimental.pallas import tpu_sc as plsc

N_NODES = 10000
N_EDGES = 320000
D = 128

NC, NS = 2, 16
NW = NC * NS
EPW = N_EDGES // NW
CHUNK = 80
NCHUNK = EPW // CHUNK
NBUF = 2
L = 16
ZC = 40
NZC = N_NODES // ZC
ZITER = -(-NZC // NS)


def _sc_body(x_hbm, edges_hbm,
             agg_out, deg_out,
             agg_sh, idx_v, rows_v, deg_v, z128_v, gsem, ssem, isem):
    cid = lax.axis_index("c")
    sid = lax.axis_index("s")
    wid = cid * NS + sid

    zeros16 = jnp.zeros((L,), jnp.float32)

    def dzero_step(k, carry):
        deg_v[pl.ds(k * L, L)] = zeros16
        return carry

    lax.fori_loop(0, N_NODES // L, dzero_step, 0)

    def rzero_step(k, carry):
        for col in range(D // L):
            z128_v[k, pl.ds(col * L, L)] = zeros16
        return carry

    lax.fori_loop(0, ZC, rzero_step, 0)

    def zero_step(k, carry):
        c = sid + k * NS

        @pl.when(c < NZC)
        def _():
            pltpu.sync_copy(z128_v, agg_sh.at[pl.ds(c * ZC, ZC)])

        return carry

    lax.fori_loop(0, ZITER, zero_step, 0)
    plsc.subcore_barrier()

    ones16 = jnp.ones((L,), jnp.float32)

    def start_idx(i, c):
        pltpu.async_copy(edges_hbm.at[wid, i], idx_v.at[c], isem.at[c])

    def wait_idx(i, c):
        pltpu.make_async_copy(
            edges_hbm.at[wid, i], idx_v.at[c], isem.at[c]).wait()

    def start_gather(b, c):
        pltpu.async_copy(x_hbm.at[idx_v.at[c, 0]], rows_v.at[b], gsem.at[b])

    def wait_gather(b, c):
        pltpu.make_async_copy(
            x_hbm.at[idx_v.at[c, 0]], rows_v.at[b], gsem.at[b]).wait()

    def start_scatter(b, c):
        pltpu.async_copy(rows_v.at[b], agg_sh.at[idx_v.at[c, 1]],
                         ssem.at[b], add=True)

    def wait_scatter(b, c):
        pltpu.make_async_copy(
            rows_v.at[b], agg_sh.at[idx_v.at[c, 1]], ssem.at[b]).wait()

    pltpu.sync_copy(edges_hbm.at[wid, 0], idx_v.at[0])
    start_idx(1, 1)
    start_gather(0, 0)

    UNROLL = 4

    def quad_step(j, carry):
        for p in range(UNROLL):
            i = UNROLL * j + p
            b, c = p % NBUF, p % 4

            @pl.when(i < NCHUNK)
            def _():
                for g in range(CHUNK // L):
                    dst16 = idx_v[c, 1, pl.ds(g * L, L)]
                    plsc.addupdate_scatter(deg_v, [dst16], ones16)
                wait_gather(b, c)
                start_scatter(b, c)

                @pl.when(i + 2 < NCHUNK)
                def _():
                    start_idx(i + 2, (p + 2) % 4)

                inext = i + 1

                @pl.when(inext < NCHUNK)
                def _():
                    b2, c2 = (p + 1) % NBUF, (p + 1) % 4

                    @pl.when(inext >= NBUF)
                    def _():
                        wait_scatter(b2, (p + 3) % 4)

                    wait_idx(inext, c2)
                    start_gather(b2, c2)

        return carry

    lax.fori_loop(0, -(-NCHUNK // UNROLL), quad_step, 0)
    wait_scatter((NCHUNK - 1) % NBUF, (NCHUNK - 1) % 4)
    wait_scatter((NCHUNK - 2) % NBUF, (NCHUNK - 2) % 4)
    plsc.subcore_barrier()

    pltpu.sync_copy(deg_v, deg_out.at[wid])

    def out_step(k, carry):
        c = sid + k * NS

        @pl.when(c < NZC)
        def _():
            r0 = c * ZC
            pltpu.sync_copy(agg_sh.at[pl.ds(r0, ZC)], z128_v)
            pltpu.sync_copy(z128_v, agg_out.at[cid, pl.ds(r0, ZC)])

        return carry

    lax.fori_loop(0, ZITER, out_step, 0)


_sc_call = pl.kernel(
    _sc_body,
    out_type=(jax.ShapeDtypeStruct((NC, N_NODES, D), jnp.float32),
              jax.ShapeDtypeStruct((NW, N_NODES), jnp.float32)),
    mesh=plsc.VectorSubcoreMesh(core_axis_name="c", subcore_axis_name="s",
                                num_cores=NC, num_subcores=NS),
    compiler_params=pltpu.CompilerParams(needs_layout_passes=False),
    scratch_types=[
        pltpu.VMEM_SHARED((N_NODES, D), jnp.float32),
        pltpu.VMEM((4, 2, CHUNK), jnp.int32),
        pltpu.VMEM((NBUF, CHUNK, D), jnp.float32),
        pltpu.VMEM((N_NODES,), jnp.float32),
        pltpu.VMEM((ZC, D), jnp.float32),
        pltpu.SemaphoreType.DMA((NBUF,)),
        pltpu.SemaphoreType.DMA((NBUF,)),
        pltpu.SemaphoreType.DMA((4,)),
    ],
)


BLK = 512


def _tc_body(x_ref, a_ref, d_ref, ws_ref, wn_ref, bs_ref, bn_ref, o_ref):
    x = x_ref[...]
    agg = a_ref[0] + a_ref[1]
    deg = jnp.sum(d_ref[...], axis=0)[:, None]
    deg = jnp.maximum(deg, 1.0)
    agg = agg / deg
    dnums = (((1,), (1,)), ((), ()))
    h = jax.lax.dot_general(x, ws_ref[...], dnums,
                            preferred_element_type=jnp.float32)
    h = h + jax.lax.dot_general(agg, wn_ref[...], dnums,
                                preferred_element_type=jnp.float32)
    h = h + bs_ref[...] + bn_ref[...]
    o_ref[...] = jnp.maximum(h, 0.0)


_tc_call = pl.pallas_call(
    _tc_body,
    grid=(-(-N_NODES // BLK),),
    in_specs=[
        pl.BlockSpec((BLK, D), lambda i: (i, 0)),
        pl.BlockSpec((NC, BLK, D), lambda i: (0, i, 0)),
        pl.BlockSpec((NW, BLK), lambda i: (0, i)),
        pl.BlockSpec((D, D), lambda i: (0, 0)),
        pl.BlockSpec((D, D), lambda i: (0, 0)),
        pl.BlockSpec((1, D), lambda i: (0, 0)),
        pl.BlockSpec((1, D), lambda i: (0, 0)),
    ],
    out_specs=pl.BlockSpec((BLK, D), lambda i: (i, 0)),
    out_shape=jax.ShapeDtypeStruct((N_NODES, D), jnp.float32),
)


def kernel(x, edge_index, W_self, b_self, W_neigh, b_neigh):
    ei = edge_index.astype(jnp.int32)
    edges = ei.reshape(2, NW, NCHUNK, CHUNK).transpose(1, 2, 0, 3)
    agg_parts, deg_parts = _sc_call(x, edges)
    return _tc_call(x, agg_parts, deg_parts, W_self, W_neigh,
                    b_self[None, :], b_neigh[None, :])
